# baseline (speedup 1.0000x reference)
import functools

import jax
import jax.numpy as jnp
from jax import lax
from jax.experimental import pallas as pl
from jax.experimental.pallas import tpu as pltpu
from jax.experimental.pallas import tpu_sc as plsc

f32 = jnp.float32
i32 = jnp.int32

N = 10000
E = 320000
IND = 128
OUTD = 64
EDGED = 16
H1 = 32

NC = 2
NS = 16
NW = NC * NS
EPW = E // NW
CH = 80
NCHUNK = EPW // CH
NPAD = 10240
RPT = NPAD // NS

E2 = E // 2
EPH = E2 // NW
CHS = 40

BE = 1000
GE = E2 // BE
BN = 2000
GN = N // BN


def _ln_k(x, g, b, eps=1e-6):
    m = jnp.mean(x, axis=-1, keepdims=True)
    v = jnp.mean((x - m) ** 2, axis=-1, keepdims=True)
    return (x - m) * lax.rsqrt(v + eps) * g + b


def _full(shape):
    return pl.BlockSpec(shape, lambda i: tuple(0 for _ in shape))


def _node_body(x_ref, dummy, g0, b0, Adt, Ast, Wst, bs, Wgt, bg,
               pd_ref, ps_ref, gate_ref, gskip_ref):
    x = x_ref[...]
    bad = x[:, 0:1] == -999.0
    x = jnp.where(bad, dummy[...], x)
    xn = _ln_k(x, g0[...], b0[...])
    pd_ref[...] = xn @ Adt[...]
    ps_ref[...] = xn @ Ast[...]
    skip = xn @ Wst[...] + bs[...]
    gate = jax.nn.sigmoid(skip @ Wgt[...] + bg[...])
    gate_ref[...] = gate
    gskip_ref[...] = gate * skip


def _node(x, *ws):
    specs = [pl.BlockSpec((BN, IND), lambda i: (i, 0))]
    specs += [_full(w.shape) for w in ws]
    return pl.pallas_call(
        _node_body,
        grid=(GN,),
        in_specs=specs,
        out_specs=[
            pl.BlockSpec((BN, H1), lambda i: (i, 0)),
            pl.BlockSpec((BN, H1), lambda i: (i, 0)),
            pl.BlockSpec((BN, OUTD), lambda i: (i, 0)),
            pl.BlockSpec((BN, OUTD), lambda i: (i, 0)),
        ],
        out_shape=[
            jax.ShapeDtypeStruct((N, H1), f32),
            jax.ShapeDtypeStruct((N, H1), f32),
            jax.ShapeDtypeStruct((N, OUTD), f32),
            jax.ShapeDtypeStruct((N, OUTD), f32),
        ],
    )(x, *ws)


def _mlp1_body(pre, ea, ge, be, W1t, b1, W2t, b2, W3t, b3, Wc1t, bc1,
               Wc2t, bc2, Aet, b1a, W1bt, b1b, W1ct, b1c, out, ee_out):
    a = ea[...]
    h = _ln_k(a, ge[...], be[...])
    h = jnp.maximum(h @ W1t[...] + b1[...], 0.0)
    h = jnp.maximum(h @ W2t[...] + b2[...], 0.0)
    enc = h @ W3t[...] + b3[...]
    cw = jnp.maximum(a @ Wc1t[...] + bc1[...], 0.0)
    w = jax.nn.sigmoid(cw @ Wc2t[...] + bc2[...])
    e = enc * w
    ee_out[...] = e
    m = jnp.maximum(pre[...][:, :H1] + e @ Aet[...] + b1a[...], 0.0)
    m = jnp.maximum(m @ W1bt[...] + b1b[...], 0.0)
    m = m @ W1ct[...] + b1c[...]
    colid = lax.broadcasted_iota(i32, (BE, IND - H1), 1)
    aug = jnp.where(colid == 0, 1.0, 0.0).astype(f32)
    out[...] = jnp.concatenate([m, aug], axis=1)


def _make_mlp1(h):
    def call(pre, ea, *ws):
        specs = [
            pl.BlockSpec((BE, IND), lambda i: (i, 0)),
            pl.BlockSpec((BE, EDGED),
                         lambda i: ((i // 5) * 10 + h * 5 + (i % 5), 0)),
        ]
        specs += [_full(w.shape) for w in ws]
        return pl.pallas_call(
            _mlp1_body,
            grid=(GE,),
            in_specs=specs,
            out_specs=[
                pl.BlockSpec((BE, IND), lambda i: (i, 0)),
                pl.BlockSpec((BE, OUTD), lambda i: (i, 0)),
            ],
            out_shape=[
                jax.ShapeDtypeStruct((E2, IND), f32),
                jax.ShapeDtypeStruct((E2, OUTD), f32),
            ],
        )(pre, ea, *ws)
    return call


_mlp1 = [_make_mlp1(h) for h in range(2)]


def _x1_body(pa, pb, pc, pd_, g1, b1, Bdt, Bst, qd_ref, qs_ref, invd_ref):
    s = pa[0] + pb[0] + pc[0] + pd_[0]
    cnt = s[:, H1:H1 + 1]
    invd = 1.0 / jnp.maximum(cnt, 1.0)
    z = _ln_k(s[:, :H1] * invd, g1[...], b1[...])
    z = jnp.where(z >= 0.0, z, 0.01 * z)
    qd_ref[...] = z @ Bdt[...]
    qs_ref[...] = z @ Bst[...]
    invd_ref[...] = invd


def _x1(s1a, s1b, g1, b1, Bdt, Bst):
    return pl.pallas_call(
        _x1_body,
        grid=(GN,),
        in_specs=[
            pl.BlockSpec((1, BN, IND), lambda i: (0, i, 0)),
            pl.BlockSpec((1, BN, IND), lambda i: (1, i, 0)),
            pl.BlockSpec((1, BN, IND), lambda i: (0, i, 0)),
            pl.BlockSpec((1, BN, IND), lambda i: (1, i, 0)),
            _full(g1.shape),
            _full(b1.shape),
            _full(Bdt.shape),
            _full(Bst.shape),
        ],
        out_specs=[
            pl.BlockSpec((BN, OUTD), lambda i: (i, 0)),
            pl.BlockSpec((BN, OUTD), lambda i: (i, 0)),
            pl.BlockSpec((BN, 1), lambda i: (i, 0)),
        ],
        out_shape=[
            jax.ShapeDtypeStruct((N, OUTD), f32),
            jax.ShapeDtypeStruct((N, OUTD), f32),
            jax.ShapeDtypeStruct((N, 1), f32),
        ],
    )(s1a, s1a, s1b, s1b, g1, b1, Bdt, Bst)


def _mlp2_body(pre, ee, Bet, b2a, W2bt, b2b, W2ct, b2c, out):
    e = ee[...]
    m = jnp.maximum(pre[...][:, :OUTD] + e @ Bet[...] + b2a[...], 0.0)
    m = jnp.maximum(m @ W2bt[...] + b2b[...], 0.0)
    m = m @ W2ct[...] + b2c[...]
    out[...] = jnp.concatenate([m, e], axis=1)


def _mlp2(pre, ee, *ws):
    specs = [
        pl.BlockSpec((BE, IND), lambda i: (i, 0)),
        pl.BlockSpec((BE, OUTD), lambda i: (i, 0)),
    ]
    specs += [_full(w.shape) for w in ws]
    return pl.pallas_call(
        _mlp2_body,
        grid=(GE,),
        in_specs=specs,
        out_specs=pl.BlockSpec((BE, IND), lambda i: (i, 0)),
        out_shape=jax.ShapeDtypeStruct((E2, IND), f32),
    )(pre, ee, *ws)


def _final_body(p2a, p2b, p2c, p2d, invd, gate, gskip, g2, b2,
                Wp1t, bp1, Wp2t, bp2, Wp3t, bp3, xfc_ref, probs_ref):
    inv = invd[...]
    s = (p2a[0] + p2b[0] + p2c[0] + p2d[0]) * inv
    x2 = _ln_k(s[:, :OUTD], g2[...], b2[...])
    x2 = jnp.maximum(x2, 0.0)
    efm = s[:, OUTD:]
    g = gate[...]
    xf = gskip[...] + (1.0 - g) * x2
    xfc = jnp.concatenate([xf, efm], axis=1)
    xfc_ref[...] = xfc
    h = xfc @ Wp1t[...] + bp1[...]
    h = jnp.where(h > 0.0, h, jnp.exp(h) - 1.0)
    h = h @ Wp2t[...] + bp2[...]
    h = jnp.where(h > 0.0, h, jnp.exp(h) - 1.0)
    probs_ref[...] = h @ Wp3t[...] + bp3[...]


def _final(s2a, s2b, invd, gate, gskip, *ws):
    specs = [
        pl.BlockSpec((1, BN, IND), lambda i: (0, i, 0)),
        pl.BlockSpec((1, BN, IND), lambda i: (1, i, 0)),
        pl.BlockSpec((1, BN, IND), lambda i: (0, i, 0)),
        pl.BlockSpec((1, BN, IND), lambda i: (1, i, 0)),
        pl.BlockSpec((BN, 1), lambda i: (i, 0)),
        pl.BlockSpec((BN, OUTD), lambda i: (i, 0)),
        pl.BlockSpec((BN, OUTD), lambda i: (i, 0)),
    ]
    specs += [_full(w.shape) for w in ws]
    return pl.pallas_call(
        _final_body,
        grid=(GN,),
        in_specs=specs,
        out_specs=[
            pl.BlockSpec((BN, 2 * OUTD), lambda i: (i, 0)),
            pl.BlockSpec((BN, 1), lambda i: (i, 0)),
        ],
        out_shape=[
            jax.ShapeDtypeStruct((N, 2 * OUTD), f32),
            jax.ShapeDtypeStruct((N, 1), f32),
        ],
    )(s2a, s2a, s2b, s2b, invd, gate, gskip, *ws)


_sc_mesh = plsc.VectorSubcoreMesh(
    core_axis_name="c", subcore_axis_name="s", num_cores=NC, num_subcores=NS)


NB = 5
GRP = NCHUNK // NB
CHG = 200
GRPG = EPH // (CHG * NB)


def _make_gather_add(W, h):
    @functools.partial(
        pl.kernel,
        out_type=jax.ShapeDtypeStruct((E2, IND), f32),
        mesh=_sc_mesh,
        scratch_types=([pltpu.VMEM((CHG,), i32)] * (2 * NB)
                       + [pltpu.VMEM((CHG, W), f32)] * NB
                       + [pltpu.SemaphoreType.DMA] * 3),
        compiler_params=pltpu.CompilerParams(use_tc_tiling_on_sc=False),
    )
    def gather_add(ta, tb, ia, ib, out, *scr):
        idxa = scr[:NB]
        idxb = scr[NB:2 * NB]
        rows = scr[2 * NB:3 * NB]
        semi, semg, semw = scr[3 * NB:]
        c = lax.axis_index("c")
        s = lax.axis_index("s")
        wid = c * NS + s

        def group(g, carry):
            offs = [(g * NB + b) * CHG for b in range(NB)]
            bi = [wid * EPW + h * EPH + o for o in offs]
            bo = [wid * EPH + o for o in offs]
            cps = []
            for b in range(NB):
                cps.append(pltpu.async_copy(ia.at[pl.ds(bi[b], CHG)],
                                            idxa[b], semi))
                cps.append(pltpu.async_copy(ib.at[pl.ds(bi[b], CHG)],
                                            idxb[b], semi))
            for cp in cps:
                cp.wait()
            cps = [pltpu.async_copy(ta.at[idxa[b]], rows[b], semg)
                   for b in range(NB)]
            for cp in cps:
                cp.wait()
            cps = [pltpu.async_copy(tb.at[idxb[b]], rows[b], semg, add=True)
                   for b in range(NB)]
            for cp in cps:
                cp.wait()
            cps = [pltpu.async_copy(
                rows[b], out.at[pl.ds(bo[b], CHG), pl.ds(0, W)], semw)
                for b in range(NB)]
            for cp in cps:
                cp.wait()
            return carry

        lax.fori_loop(0, GRPG, group, 0)

    return gather_add


_sc_gather32 = [_make_gather_add(H1, h) for h in range(2)]
_sc_gather64 = [_make_gather_add(OUTD, h) for h in range(2)]


NBS = 4
NGPAIR = 31


def _make_scatter(h):
    @functools.partial(
        pl.kernel,
        out_type=jax.ShapeDtypeStruct((NC, NPAD, IND), f32),
        mesh=_sc_mesh,
        scratch_types=([pltpu.VMEM((CHS,), i32)] * NBS
                       + [pltpu.VMEM((CHS, IND), f32)] * NBS
                       + [pltpu.VMEM_SHARED((NPAD, IND), f32)]
                       + [pltpu.SemaphoreType.DMA] * 4),
    )
    def scatter(vals, dsti, zer, out, *scr):
        idx = scr[:NBS]
        rows = scr[NBS:2 * NBS]
        acc = scr[2 * NBS]
        semia, semib, semsa, semsb = scr[2 * NBS + 1:]
        c = lax.axis_index("c")
        s = lax.axis_index("s")
        r0 = s * RPT
        pltpu.sync_copy(zer.at[pl.ds(r0, RPT)], acc.at[pl.ds(r0, RPT)])
        plsc.subcore_barrier()
        wid = c * NS + s
        base_i = wid * EPW + h * EPH
        base_v = wid * EPH

        def fire_loads(g, sl, sem):
            cps = []
            for k in range(2):
                o = (g * 2 + k) * CHS
                cps.append(pltpu.async_copy(
                    dsti.at[pl.ds(base_i + o, CHS)], idx[sl + k], sem))
                cps.append(pltpu.async_copy(
                    vals.at[pl.ds(base_v + o, CHS)], rows[sl + k], sem))
            return cps

        def drain_loads(sl, sem):
            for k in range(2):
                pltpu.make_async_copy(dsti.at[pl.ds(base_i, CHS)],
                                      idx[sl + k], sem).wait()
                pltpu.make_async_copy(vals.at[pl.ds(base_v, CHS)],
                                      rows[sl + k], sem).wait()

        def fire_scats(sl, sem):
            return [pltpu.async_copy(rows[sl + k], acc.at[idx[sl + k]],
                                     sem, add=True) for k in range(2)]

        fire_loads(0, 0, semia)

        def pair(gg, carry):
            g0 = 2 * gg
            cps_b = fire_loads(g0 + 1, 2, semib)
            drain_loads(0, semia)
            sa = fire_scats(0, semsa)
            for cp in sa:
                cp.wait()

            @pl.when(gg + 1 < NGPAIR)
            def _():
                fire_loads(g0 + 2, 0, semia)

            for cp in cps_b:
                cp.wait()
            sb = fire_scats(2, semsb)
            for cp in sb:
                cp.wait()
            return carry

        lax.fori_loop(0, NGPAIR, pair, 0)

        o = NGPAIR * 2 * 2 * CHS
        pltpu.sync_copy(dsti.at[pl.ds(base_i + o, CHS)], idx[0])
        pltpu.sync_copy(vals.at[pl.ds(base_v + o, CHS)], rows[0])
        pltpu.sync_copy(rows[0], acc.at[idx[0]], add=True)
        plsc.subcore_barrier()
        pltpu.sync_copy(acc.at[pl.ds(r0, RPT)], out.at[c, pl.ds(r0, RPT)])

    return scatter


_sc_scatter = [_make_scatter(h) for h in range(2)]


def kernel(x_in, edge_index, edge_attr, params):
    p = params
    src = edge_index[0, 0]
    dst = edge_index[0, 1]
    x = x_in[0]
    ea = edge_attr[0]

    def r(v):
        return v.reshape(1, -1)

    Adt = p['Wm1a'][:, :IND].T
    Ast = p['Wm1a'][:, IND:2 * IND].T
    Aet = p['Wm1a'][:, 2 * IND:].T
    Bdt = p['Wm2a'][:, :H1].T
    Bst = p['Wm2a'][:, H1:2 * H1].T
    Bet = p['Wm2a'][:, 2 * H1:].T
    zer = jnp.zeros((NPAD, IND), f32)

    pd, ps, gate, gskip = _node(x, r(p['dummy']), r(p['g0']), r(p['b0']),
                                Adt, Ast,
                                p['Wskip'].T, r(p['bskip']),
                                p['Wg'].T, r(p['bg']))
    enc_ws = (r(p['ge']), r(p['be']),
              p['We1'].T, r(p['be1']), p['We2'].T, r(p['be2']),
              p['We3'].T, r(p['be3']),
              p['Wc1'].T, r(p['bc1']), p['Wc2'].T, r(p['bc2']),
              Aet, r(p['bm1a']),
              p['Wm1b'].T, r(p['bm1b']), p['Wm1c'].T, r(p['bm1c']))
    preA = _sc_gather32[0](pd, ps, dst, src)
    preB = _sc_gather32[1](pd, ps, dst, src)
    m1A, eeA = _mlp1[0](preA, ea, *enc_ws)
    m1B, eeB = _mlp1[1](preB, ea, *enc_ws)
    s1A = _sc_scatter[0](m1A, dst, zer)
    s1B = _sc_scatter[1](m1B, dst, zer)
    qd, qs, invd = _x1(s1A, s1B, r(p['g1']), r(p['b1']), Bdt, Bst)
    q2A = _sc_gather64[0](qd, qs, dst, src)
    q2B = _sc_gather64[1](qd, qs, dst, src)
    m2_ws = (Bet, r(p['bm2a']),
             p['Wm2b'].T, r(p['bm2b']), p['Wm2c'].T, r(p['bm2c']))
    m2A = _mlp2(q2A, eeA, *m2_ws)
    m2B = _mlp2(q2B, eeB, *m2_ws)
    s2A = _sc_scatter[0](m2A, dst, zer)
    s2B = _sc_scatter[1](m2B, dst, zer)
    xfc, probs = _final(s2A, s2B, invd, gate, gskip,
                        r(p['g2']), r(p['b2']),
                        p['Wp1'].T, r(p['bp1']), p['Wp2'].T, r(p['bp2']),
                        p['Wp3'].T, r(p['bp3']))
    return (xfc[None], probs[None], jnp.zeros((1,), f32))


# speedup vs baseline: 1.3927x; 1.3927x over previous
import functools

import jax
import jax.numpy as jnp
from jax import lax
from jax.experimental import pallas as pl
from jax.experimental.pallas import tpu as pltpu
from jax.experimental.pallas import tpu_sc as plsc

f32 = jnp.float32
i32 = jnp.int32

N = 10000
E = 320000
IND = 128
OUTD = 64
EDGED = 16
H1 = 32

NC = 2
NS = 16
NW = NC * NS
EPW = E // NW
CH = 80
NCHUNK = EPW // CH
NPAD = 10240
RPT = NPAD // NS

BE = 4000
GE = E // BE
BN = 2000
GN = N // BN


def _ln_k(x, g, b, eps=1e-6):
    m = jnp.mean(x, axis=-1, keepdims=True)
    v = jnp.mean((x - m) ** 2, axis=-1, keepdims=True)
    return (x - m) * lax.rsqrt(v + eps) * g + b


def _full(shape):
    return pl.BlockSpec(shape, lambda i: tuple(0 for _ in shape))


def _node_body(x_ref, dummy, g0, b0, Adt, Ast, Wst, bs, Wgt, bg,
               pd_ref, ps_ref, gate_ref, gskip_ref):
    x = x_ref[...]
    bad = x[:, 0:1] == -999.0
    x = jnp.where(bad, dummy[...], x)
    xn = _ln_k(x, g0[...], b0[...])
    pd_ref[...] = xn @ Adt[...]
    ps_ref[...] = xn @ Ast[...]
    skip = xn @ Wst[...] + bs[...]
    gate = jax.nn.sigmoid(skip @ Wgt[...] + bg[...])
    gate_ref[...] = gate
    gskip_ref[...] = gate * skip


def _node(x, *ws):
    specs = [pl.BlockSpec((BN, IND), lambda i: (i, 0))]
    specs += [_full(w.shape) for w in ws]
    return pl.pallas_call(
        _node_body,
        grid=(GN,),
        in_specs=specs,
        out_specs=[
            pl.BlockSpec((BN, H1), lambda i: (i, 0)),
            pl.BlockSpec((BN, H1), lambda i: (i, 0)),
            pl.BlockSpec((BN, OUTD), lambda i: (i, 0)),
            pl.BlockSpec((BN, OUTD), lambda i: (i, 0)),
        ],
        out_shape=[
            jax.ShapeDtypeStruct((N, H1), f32),
            jax.ShapeDtypeStruct((N, H1), f32),
            jax.ShapeDtypeStruct((N, OUTD), f32),
            jax.ShapeDtypeStruct((N, OUTD), f32),
        ],
    )(x, *ws)


def _mlp1_body(pre, ea, ge, be, W1t, b1, W2t, b2, W3t, b3, Wc1t, bc1,
               Wc2t, bc2, Aet, b1a, W1bt, b1b, W1ct, b1c, out, ee_out):
    a = ea[...]
    h = _ln_k(a, ge[...], be[...])
    h = jnp.maximum(h @ W1t[...] + b1[...], 0.0)
    h = jnp.maximum(h @ W2t[...] + b2[...], 0.0)
    enc = h @ W3t[...] + b3[...]
    cw = jnp.maximum(a @ Wc1t[...] + bc1[...], 0.0)
    w = jax.nn.sigmoid(cw @ Wc2t[...] + bc2[...])
    e = enc * w
    ee_out[...] = e
    m = jnp.maximum(pre[...][:, :H1] + e @ Aet[...] + b1a[...], 0.0)
    m = jnp.maximum(m @ W1bt[...] + b1b[...], 0.0)
    m = m @ W1ct[...] + b1c[...]
    colid = lax.broadcasted_iota(i32, (BE, IND - H1), 1)
    aug = jnp.where(colid == 0, 1.0, 0.0).astype(f32)
    out[...] = jnp.concatenate([m, aug], axis=1)


def _mlp1(pre, ea, *ws):
    specs = [
        pl.BlockSpec((BE, IND), lambda i: (i, 0)),
        pl.BlockSpec((BE, EDGED), lambda i: (i, 0)),
    ]
    specs += [_full(w.shape) for w in ws]
    return pl.pallas_call(
        _mlp1_body,
        grid=(GE,),
        in_specs=specs,
        out_specs=[
            pl.BlockSpec((BE, IND), lambda i: (i, 0)),
            pl.BlockSpec((BE, OUTD), lambda i: (i, 0)),
        ],
        out_shape=[
            jax.ShapeDtypeStruct((E, IND), f32),
            jax.ShapeDtypeStruct((E, OUTD), f32),
        ],
    )(pre, ea, *ws)


def _x1_body(pa, pb, g1, b1, Bdt, Bst, qd_ref, qs_ref, invd_ref):
    s = pa[0] + pb[0]
    cnt = s[:, H1:H1 + 1]
    invd = 1.0 / jnp.maximum(cnt, 1.0)
    z = _ln_k(s[:, :H1] * invd, g1[...], b1[...])
    z = jnp.where(z >= 0.0, z, 0.01 * z)
    qd_ref[...] = z @ Bdt[...]
    qs_ref[...] = z @ Bst[...]
    invd_ref[...] = invd


def _x1(s1, g1, b1, Bdt, Bst):
    return pl.pallas_call(
        _x1_body,
        grid=(GN,),
        in_specs=[
            pl.BlockSpec((1, BN, IND), lambda i: (0, i, 0)),
            pl.BlockSpec((1, BN, IND), lambda i: (1, i, 0)),
            _full(g1.shape),
            _full(b1.shape),
            _full(Bdt.shape),
            _full(Bst.shape),
        ],
        out_specs=[
            pl.BlockSpec((BN, OUTD), lambda i: (i, 0)),
            pl.BlockSpec((BN, OUTD), lambda i: (i, 0)),
            pl.BlockSpec((BN, 1), lambda i: (i, 0)),
        ],
        out_shape=[
            jax.ShapeDtypeStruct((N, OUTD), f32),
            jax.ShapeDtypeStruct((N, OUTD), f32),
            jax.ShapeDtypeStruct((N, 1), f32),
        ],
    )(s1, s1, g1, b1, Bdt, Bst)


def _mlp2_body(pre, ee, Bet, b2a, W2bt, b2b, W2ct, b2c, out):
    e = ee[...]
    m = jnp.maximum(pre[...][:, :OUTD] + e @ Bet[...] + b2a[...], 0.0)
    m = jnp.maximum(m @ W2bt[...] + b2b[...], 0.0)
    m = m @ W2ct[...] + b2c[...]
    out[...] = jnp.concatenate([m, e], axis=1)


def _mlp2(pre, ee, *ws):
    specs = [
        pl.BlockSpec((BE, IND), lambda i: (i, 0)),
        pl.BlockSpec((BE, OUTD), lambda i: (i, 0)),
    ]
    specs += [_full(w.shape) for w in ws]
    return pl.pallas_call(
        _mlp2_body,
        grid=(GE,),
        in_specs=specs,
        out_specs=pl.BlockSpec((BE, IND), lambda i: (i, 0)),
        out_shape=jax.ShapeDtypeStruct((E, IND), f32),
    )(pre, ee, *ws)


def _final_body(p2a, p2b, invd, gate, gskip, g2, b2,
                Wp1t, bp1, Wp2t, bp2, Wp3t, bp3, xfc_ref, probs_ref):
    inv = invd[...]
    s = (p2a[0] + p2b[0]) * inv
    x2 = _ln_k(s[:, :OUTD], g2[...], b2[...])
    x2 = jnp.maximum(x2, 0.0)
    efm = s[:, OUTD:]
    g = gate[...]
    xf = gskip[...] + (1.0 - g) * x2
    xfc = jnp.concatenate([xf, efm], axis=1)
    xfc_ref[...] = xfc
    h = xfc @ Wp1t[...] + bp1[...]
    h = jnp.where(h > 0.0, h, jnp.exp(h) - 1.0)
    h = h @ Wp2t[...] + bp2[...]
    h = jnp.where(h > 0.0, h, jnp.exp(h) - 1.0)
    probs_ref[...] = h @ Wp3t[...] + bp3[...]


def _final(s2, invd, gate, gskip, *ws):
    specs = [
        pl.BlockSpec((1, BN, IND), lambda i: (0, i, 0)),
        pl.BlockSpec((1, BN, IND), lambda i: (1, i, 0)),
        pl.BlockSpec((BN, 1), lambda i: (i, 0)),
        pl.BlockSpec((BN, OUTD), lambda i: (i, 0)),
        pl.BlockSpec((BN, OUTD), lambda i: (i, 0)),
    ]
    specs += [_full(w.shape) for w in ws]
    return pl.pallas_call(
        _final_body,
        grid=(GN,),
        in_specs=specs,
        out_specs=[
            pl.BlockSpec((BN, 2 * OUTD), lambda i: (i, 0)),
            pl.BlockSpec((BN, 1), lambda i: (i, 0)),
        ],
        out_shape=[
            jax.ShapeDtypeStruct((N, 2 * OUTD), f32),
            jax.ShapeDtypeStruct((N, 1), f32),
        ],
    )(s2, s2, invd, gate, gskip, *ws)


_sc_mesh = plsc.VectorSubcoreMesh(
    core_axis_name="c", subcore_axis_name="s", num_cores=NC, num_subcores=NS)


NB = 5
GRP = NCHUNK // NB
CHG = 200
GRPG = EPW // (CHG * NB)


def _make_gather_add(W):
    @functools.partial(
        pl.kernel,
        out_type=jax.ShapeDtypeStruct((E, IND), f32),
        mesh=_sc_mesh,
        scratch_types=([pltpu.VMEM((CHG,), i32)] * (2 * NB)
                       + [pltpu.VMEM((CHG, W), f32)] * NB
                       + [pltpu.SemaphoreType.DMA] * 3),
        compiler_params=pltpu.CompilerParams(use_tc_tiling_on_sc=False),
    )
    def gather_add(ta, tb, ia, ib, out, *scr):
        idxa = scr[:NB]
        idxb = scr[NB:2 * NB]
        rows = scr[2 * NB:3 * NB]
        semi, semg, semw = scr[3 * NB:]
        c = lax.axis_index("c")
        s = lax.axis_index("s")
        wid = c * NS + s

        def fire_idx(g):
            for b in range(NB):
                base = wid * EPW + (g * NB + b) * CHG
                pltpu.async_copy(ia.at[pl.ds(base, CHG)], idxa[b], semi)
                pltpu.async_copy(ib.at[pl.ds(base, CHG)], idxb[b], semi)

        def drain_idx():
            for b in range(NB):
                pltpu.make_async_copy(ia.at[pl.ds(0, CHG)],
                                      idxa[b], semi).wait()
                pltpu.make_async_copy(ib.at[pl.ds(0, CHG)],
                                      idxb[b], semi).wait()

        fire_idx(0)

        def group(g, carry):
            bases = [wid * EPW + (g * NB + b) * CHG for b in range(NB)]
            drain_idx()
            cps = [pltpu.async_copy(ta.at[idxa[b]], rows[b], semg)
                   for b in range(NB)]
            for cp in cps:
                cp.wait()
            cps = [pltpu.async_copy(tb.at[idxb[b]], rows[b], semg, add=True)
                   for b in range(NB)]
            for cp in cps:
                cp.wait()

            cps = [pltpu.async_copy(
                rows[b], out.at[pl.ds(bases[b], CHG), pl.ds(0, W)], semw)
                for b in range(NB)]

            @pl.when(g + 1 < GRPG)
            def _():
                fire_idx(g + 1)

            for cp in cps:
                cp.wait()
            return carry

        lax.fori_loop(0, GRPG, group, 0)

    return gather_add


_sc_gather_add32 = _make_gather_add(H1)
_sc_gather_add64 = _make_gather_add(OUTD)


NBS = 4
NGPAIR = 31


@functools.partial(
    pl.kernel,
    out_type=jax.ShapeDtypeStruct((NC, NPAD, IND), f32),
    mesh=_sc_mesh,
    scratch_types=([pltpu.VMEM((CH,), i32)] * NBS
                   + [pltpu.VMEM((CH, IND), f32)] * NBS
                   + [pltpu.VMEM_SHARED((NPAD, IND), f32)]
                   + [pltpu.SemaphoreType.DMA] * 4),
)
def _sc_scatter128(vals, dsti, zer, out, *scr):
    idx = scr[:NBS]
    rows = scr[NBS:2 * NBS]
    acc = scr[2 * NBS]
    semia, semib, semsa, semsb = scr[2 * NBS + 1:]
    c = lax.axis_index("c")
    s = lax.axis_index("s")
    r0 = s * RPT
    pltpu.sync_copy(zer.at[pl.ds(r0, RPT)], acc.at[pl.ds(r0, RPT)])
    plsc.subcore_barrier()
    wid = c * NS + s
    base0 = wid * EPW

    def fire_loads(g, sl, sem):
        cps = []
        for k in range(2):
            b = g * 2 + k
            cps.append(pltpu.async_copy(
                dsti.at[pl.ds(base0 + b * CH, CH)], idx[sl + k], sem))
            cps.append(pltpu.async_copy(
                vals.at[pl.ds(base0 + b * CH, CH)], rows[sl + k], sem))
        return cps

    def drain_loads(sl, sem):
        for k in range(2):
            pltpu.make_async_copy(dsti.at[pl.ds(base0, CH)],
                                  idx[sl + k], sem).wait()
            pltpu.make_async_copy(vals.at[pl.ds(base0, CH)],
                                  rows[sl + k], sem).wait()

    def fire_scats(sl, sem):
        return [pltpu.async_copy(rows[sl + k], acc.at[idx[sl + k]], sem,
                                 add=True) for k in range(2)]

    fire_loads(0, 0, semia)

    def pair(gg, carry):
        g0 = 2 * gg
        cps_b = fire_loads(g0 + 1, 2, semib)
        drain_loads(0, semia)
        sa = fire_scats(0, semsa)
        for cp in sa:
            cp.wait()

        @pl.when(gg + 1 < NGPAIR)
        def _():
            fire_loads(g0 + 2, 0, semia)

        for cp in cps_b:
            cp.wait()
        sb = fire_scats(2, semsb)
        for cp in sb:
            cp.wait()
        return carry

    lax.fori_loop(0, NGPAIR, pair, 0)

    tail = base0 + NGPAIR * 2 * 2 * CH
    pltpu.sync_copy(dsti.at[pl.ds(tail, CH)], idx[0])
    pltpu.sync_copy(vals.at[pl.ds(tail, CH)], rows[0])
    pltpu.sync_copy(rows[0], acc.at[idx[0]], add=True)
    plsc.subcore_barrier()
    pltpu.sync_copy(acc.at[pl.ds(r0, RPT)], out.at[c, pl.ds(r0, RPT)])


def kernel(x_in, edge_index, edge_attr, params):
    p = params
    src = edge_index[0, 0]
    dst = edge_index[0, 1]
    x = x_in[0]
    ea = edge_attr[0]

    def r(v):
        return v.reshape(1, -1)

    Adt = p['Wm1a'][:, :IND].T
    Ast = p['Wm1a'][:, IND:2 * IND].T
    Aet = p['Wm1a'][:, 2 * IND:].T
    Bdt = p['Wm2a'][:, :H1].T
    Bst = p['Wm2a'][:, H1:2 * H1].T
    Bet = p['Wm2a'][:, 2 * H1:].T
    zer = jnp.zeros((NPAD, IND), f32)

    pd, ps, gate, gskip = _node(x, r(p['dummy']), r(p['g0']), r(p['b0']),
                                Adt, Ast,
                                p['Wskip'].T, r(p['bskip']),
                                p['Wg'].T, r(p['bg']))
    m1pre = _sc_gather_add32(pd, ps, dst, src)
    m1, e_enc = _mlp1(m1pre, ea,
                      r(p['ge']), r(p['be']),
                      p['We1'].T, r(p['be1']), p['We2'].T, r(p['be2']),
                      p['We3'].T, r(p['be3']),
                      p['Wc1'].T, r(p['bc1']), p['Wc2'].T, r(p['bc2']),
                      Aet, r(p['bm1a']),
                      p['Wm1b'].T, r(p['bm1b']), p['Wm1c'].T, r(p['bm1c']))
    s1 = _sc_scatter128(m1, dst, zer)
    qd, qs, invd = _x1(s1, r(p['g1']), r(p['b1']), Bdt, Bst)
    m2pre = _sc_gather_add64(qd, qs, dst, src)
    m2 = _mlp2(m2pre, e_enc, Bet, r(p['bm2a']),
               p['Wm2b'].T, r(p['bm2b']), p['Wm2c'].T, r(p['bm2c']))
    s2 = _sc_scatter128(m2, dst, zer)
    xfc, probs = _final(s2, invd, gate, gskip,
                        r(p['g2']), r(p['b2']),
                        p['Wp1'].T, r(p['bp1']), p['Wp2'].T, r(p['bp2']),
                        p['Wp3'].T, r(p['bp3']))
    return (xfc[None], probs[None], jnp.zeros((1,), f32))
